# Initial kernel scaffold; baseline (speedup 1.0000x reference)
import jax
import jax.numpy as jnp
from jax.experimental import pallas as pl


def kernel(input, target):
    raise NotImplementedError("write your pallas kernel here")


# speedup vs baseline: 4.5552x; 4.5552x over previous
import functools

import jax
import jax.numpy as jnp
from jax import lax
from jax.experimental import pallas as pl
from jax.experimental.pallas import tpu as pltpu
from jax.experimental.pallas import tpu_sc as plsc

B, L = 1024, 256
NC, NS = 2, 16
NW = NC * NS
RPW = B // NW
LANES = 16
NCH = L // LANES

_mesh = plsc.VectorSubcoreMesh(core_axis_name="c", subcore_axis_name="s")


@functools.partial(
    pl.kernel,
    mesh=_mesh,
    compiler_params=pltpu.CompilerParams(needs_layout_passes=False),
    out_type=jax.ShapeDtypeStruct((NW, LANES), jnp.float32),
    scratch_types=[
        pltpu.VMEM((RPW, L), jnp.float32),
        pltpu.VMEM((RPW, L), jnp.int32),
        pltpu.VMEM((LANES,), jnp.float32),
    ],
)
def _bpmll_partials(x_hbm, t_hbm, out_hbm, x_v, t_v, o_v):
    wid = lax.axis_index("s") * NC + lax.axis_index("c")
    base = wid * RPW
    pltpu.sync_copy(x_hbm.at[pl.ds(base, RPW)], x_v)
    pltpu.sync_copy(t_hbm.at[pl.ds(base, RPW)], t_v)

    def row_body(r, acc):
        spos = jnp.zeros((LANES,), jnp.float32)
        sneg = jnp.zeros((LANES,), jnp.float32)
        cnt = jnp.zeros((LANES,), jnp.float32)
        one = jnp.ones((LANES,), jnp.float32)
        zero = jnp.zeros((LANES,), jnp.float32)
        for j in range(NCH):
            xv = x_v[r, pl.ds(j * LANES, LANES)]
            tv = t_v[r, pl.ds(j * LANES, LANES)]
            pos = tv == 1
            ex = jnp.exp(xv)
            spos = spos + jnp.where(pos, one / ex, zero)
            sneg = sneg + jnp.where(pos, zero, ex)
            cnt = cnt + jnp.where(pos, one, zero)
        npos = jnp.sum(cnt)
        num = jnp.sum(sneg) * jnp.sum(spos)
        den = npos * (jnp.float32(L) - npos)
        numv = jnp.full((LANES,), num, jnp.float32)
        denv = jnp.full((LANES,), den, jnp.float32)
        return acc + numv / denv

    acc = lax.fori_loop(0, RPW, row_body, jnp.zeros((LANES,), jnp.float32))
    lane = lax.iota(jnp.int32, LANES)
    o_v[...] = jnp.where(lane == 0, acc * jnp.float32(1.0 / B), jnp.float32(0.0))
    pltpu.sync_copy(o_v, out_hbm.at[wid])


def kernel(input, target):
    parts = _bpmll_partials(input, target.astype(jnp.int32))
    return parts.sum()
